# baseline (speedup 1.0000x reference)
import functools

import jax
import jax.numpy as jnp
from jax import lax
from jax.experimental import pallas as pl
from jax.experimental.pallas import tpu as pltpu
from jax.experimental.pallas import tpu_sc as plsc

D_MODEL = 256
N_SITES = 512
BATCH = 1024

_NUM_CORES = 2
_NUM_SUBCORES = 16
_LANES = 16
_NW = _NUM_CORES * _NUM_SUBCORES
_ROWS_PER_W = BATCH // _NW
_GCH = 48
_ECH = 80
_PAIR = _GCH + _ECH
_PPR = N_SITES // _PAIR
_NPAIR = _ROWS_PER_W * _PPR
_TBLK = 8
_REP = 1024


def _make_sc_embed():
    mesh = plsc.VectorSubcoreMesh(core_axis_name="c", subcore_axis_name="s")

    @functools.partial(
        pl.kernel,
        mesh=mesh,
        out_type=jax.ShapeDtypeStruct((BATCH, N_SITES, D_MODEL), jnp.float32),
        scratch_types=[
            pltpu.VMEM((_ROWS_PER_W, 2 * N_SITES), jnp.int32),
            pltpu.VMEM((4, D_MODEL), jnp.float32),
        ]
        + [pltpu.VMEM((_GCH,), jnp.int32) for _ in range(2)]
        + [pltpu.VMEM((_GCH, D_MODEL), jnp.float32) for _ in range(2)]
        + [pltpu.VMEM((_ECH, D_MODEL), jnp.float32) for _ in range(2)]
        + [pltpu.SemaphoreType.DMA for _ in range(6)],
    )
    def sc_embed(n_hbm, trep_hbm, combo_hbm, out_hbm, slab_v, table_v, *bufs):
        tok_v = bufs[0:2]
        gbuf = bufs[2:4]
        ebuf = bufs[4:6]
        g_sem = bufs[6:8]
        gw_sem = bufs[8:10]
        ew_sem = bufs[10:12]
        wid = lax.axis_index("s") * _NUM_CORES + lax.axis_index("c")

        pltpu.sync_copy(combo_hbm, table_v)
        pltpu.sync_copy(n_hbm.at[pl.ds(wid * _ROWS_PER_W, _ROWS_PER_W)], slab_v)

        def pos(p):
            return p // _PPR, (p % _PPR) * _PAIR

        def fire_gather(p, q):
            r, o = pos(p)
            for i in range(_GCH // _LANES):
                dn = slab_v[r, pl.ds(o + i * _LANES, _LANES)]
                up = slab_v[r, pl.ds(N_SITES + o + i * _LANES, _LANES)]
                rep = (lax.iota(jnp.int32, _LANES)
                       + (p * _GCH + i * _LANES)) & (_REP - 1)
                tok_v[q][pl.ds(i * _LANES, _LANES)] = up + dn + dn + rep * 4
            pltpu.async_copy(trep_hbm.at[tok_v[q]], gbuf[q], g_sem[q])

        def wait_gather(q):
            pltpu.make_async_copy(
                trep_hbm.at[tok_v[q]], gbuf[q], g_sem[q]).wait()

        def gout(p):
            r, o = pos(p)
            return out_hbm.at[wid * _ROWS_PER_W + r, pl.ds(o, _GCH)]

        def eout(p):
            r, o = pos(p)
            return out_hbm.at[wid * _ROWS_PER_W + r, pl.ds(o + _GCH, _ECH)]

        def expand(p, q):
            r, o = pos(p)
            eo = o + _GCH
            for b in range(_ECH // _TBLK):
                va = (b * _TBLK // _LANES) * _LANES
                j0 = (b * _TBLK) % _LANES
                dnv = slab_v[r, pl.ds(eo + va, _LANES)].astype(jnp.float32)
                upv = slab_v[r, pl.ds(N_SITES + eo + va, _LANES)].astype(
                    jnp.float32)
                df = [jnp.full((_LANES,), dnv[j0 + j], jnp.float32)
                      for j in range(_TBLK)]
                uf = [jnp.full((_LANES,), upv[j0 + j], jnp.float32)
                      for j in range(_TBLK)]

                def i_body(i, carry, b=b, q=q, df=df, uf=uf):
                    s = i * _LANES
                    p0 = table_v[0, pl.ds(s, _LANES)]
                    p1 = table_v[1, pl.ds(s, _LANES)]
                    p2 = table_v[2, pl.ds(s, _LANES)]
                    p3 = table_v[3, pl.ds(s, _LANES)]
                    for j in range(_TBLK):
                        a = p0 + uf[j] * p1
                        bb = p2 + uf[j] * p3
                        ebuf[q][b * _TBLK + j, pl.ds(s, _LANES)] = (
                            a + df[j] * bb)
                    return carry

                lax.fori_loop(0, D_MODEL // _LANES, i_body, 0)

        fire_gather(0, 0)

        def pair_body(g2, carry):
            for q in range(2):
                p = g2 * 2 + q

                def drain_e(p=p, q=q):
                    pltpu.make_async_copy(ebuf[q], eout(p - 2), ew_sem[q]).wait()

                pl.when(g2 > 0)(drain_e)
                expand(p, q)
                pltpu.async_copy(ebuf[q], eout(p), ew_sem[q])

                wait_gather(q)
                pltpu.async_copy(gbuf[q], gout(p), gw_sem[q])

                def drain_g(p=p, q=q):
                    pltpu.make_async_copy(
                        gbuf[1 - q], gout(p - 1), gw_sem[1 - q]).wait()

                def nxt(p=p, q=q):
                    fire_gather(p + 1, 1 - q)

                if q == 1:
                    drain_g()
                    pl.when(g2 < _NPAIR // 2 - 1)(nxt)
                else:
                    pl.when(g2 > 0)(drain_g)
                    nxt()
            return carry

        lax.fori_loop(0, _NPAIR // 2, pair_body, 0)
        pltpu.make_async_copy(gbuf[1], gout(_NPAIR - 1), gw_sem[1]).wait()
        pltpu.make_async_copy(ebuf[0], eout(_NPAIR - 2), ew_sem[0]).wait()
        pltpu.make_async_copy(ebuf[1], eout(_NPAIR - 1), ew_sem[1]).wait()

    return sc_embed


_sc_embed = _make_sc_embed()


def kernel(n_flat, embed_table):
    n = jnp.asarray(n_flat, jnp.int32)
    t = jnp.asarray(embed_table, jnp.float32)
    table_rep = jnp.tile(t, (_REP, 1))
    combos = jnp.stack([t[0], t[1] - t[0], t[2] - t[0],
                        t[3] - t[1] - t[2] + t[0]])
    return _sc_embed(n, table_rep, combos)


# speedup vs baseline: 1.1178x; 1.1178x over previous
import functools

import jax
import jax.numpy as jnp
from jax import lax
from jax.experimental import pallas as pl
from jax.experimental.pallas import tpu as pltpu
from jax.experimental.pallas import tpu_sc as plsc

D_MODEL = 256
N_SITES = 512
BATCH = 1024

_NUM_CORES = 2
_NUM_SUBCORES = 16
_LANES = 16
_NW = _NUM_CORES * _NUM_SUBCORES
_ROWS_PER_W = BATCH // _NW
_GCH = 64
_ECH = 64
_PAIR = _GCH + _ECH
_PPR = N_SITES // _PAIR
_NPAIR = _ROWS_PER_W * _PPR
_TBLK = 16
_REP = 1024


def _make_sc_embed():
    mesh = plsc.VectorSubcoreMesh(core_axis_name="c", subcore_axis_name="s")

    @functools.partial(
        pl.kernel,
        mesh=mesh,
        out_type=jax.ShapeDtypeStruct((BATCH, N_SITES, D_MODEL), jnp.float32),
        scratch_types=[
            pltpu.VMEM((_ROWS_PER_W, 2 * N_SITES), jnp.int32),
            pltpu.VMEM((4, D_MODEL), jnp.float32),
        ]
        + [pltpu.VMEM((_GCH,), jnp.int32) for _ in range(2)]
        + [pltpu.VMEM((_GCH, D_MODEL), jnp.float32) for _ in range(2)]
        + [pltpu.VMEM((_ECH, D_MODEL), jnp.float32) for _ in range(2)]
        + [pltpu.SemaphoreType.DMA for _ in range(6)],
    )
    def sc_embed(n_hbm, trep_hbm, combo_hbm, out_hbm, slab_v, table_v, *bufs):
        tok_v = bufs[0:2]
        gbuf = bufs[2:4]
        ebuf = bufs[4:6]
        g_sem = bufs[6:8]
        gw_sem = bufs[8:10]
        ew_sem = bufs[10:12]
        wid = lax.axis_index("s") * _NUM_CORES + lax.axis_index("c")

        pltpu.sync_copy(combo_hbm, table_v)
        pltpu.sync_copy(n_hbm.at[pl.ds(wid * _ROWS_PER_W, _ROWS_PER_W)], slab_v)

        def pos(p):
            return p // _PPR, (p % _PPR) * _PAIR

        def fire_gather(p, q):
            r, o = pos(p)
            for i in range(_GCH // _LANES):
                dn = slab_v[r, pl.ds(o + i * _LANES, _LANES)]
                up = slab_v[r, pl.ds(N_SITES + o + i * _LANES, _LANES)]
                rep = (lax.iota(jnp.int32, _LANES)
                       + (p * _GCH + i * _LANES)) & (_REP - 1)
                tok_v[q][pl.ds(i * _LANES, _LANES)] = up + dn + dn + rep * 4
            pltpu.async_copy(trep_hbm.at[tok_v[q]], gbuf[q], g_sem[q])

        def wait_gather(q):
            pltpu.make_async_copy(
                trep_hbm.at[tok_v[q]], gbuf[q], g_sem[q]).wait()

        def gout(p):
            r, o = pos(p)
            return out_hbm.at[wid * _ROWS_PER_W + r, pl.ds(o, _GCH)]

        def eout(p):
            r, o = pos(p)
            return out_hbm.at[wid * _ROWS_PER_W + r, pl.ds(o + _GCH, _ECH)]

        def expand(p, q):
            r, o = pos(p)
            eo = o + _GCH
            for b in range(_ECH // _TBLK):
                va = (b * _TBLK // _LANES) * _LANES
                j0 = (b * _TBLK) % _LANES
                dnv = slab_v[r, pl.ds(eo + va, _LANES)].astype(jnp.float32)
                upv = slab_v[r, pl.ds(N_SITES + eo + va, _LANES)].astype(
                    jnp.float32)
                df = [jnp.full((_LANES,), dnv[j0 + j], jnp.float32)
                      for j in range(_TBLK)]
                uf = [jnp.full((_LANES,), upv[j0 + j], jnp.float32)
                      for j in range(_TBLK)]

                def i_body(i, carry, b=b, q=q, df=df, uf=uf):
                    s = i * _LANES
                    p0 = table_v[0, pl.ds(s, _LANES)]
                    p1 = table_v[1, pl.ds(s, _LANES)]
                    p2 = table_v[2, pl.ds(s, _LANES)]
                    p3 = table_v[3, pl.ds(s, _LANES)]
                    for j in range(_TBLK):
                        a = p0 + uf[j] * p1
                        bb = p2 + uf[j] * p3
                        ebuf[q][b * _TBLK + j, pl.ds(s, _LANES)] = (
                            a + df[j] * bb)
                    return carry

                lax.fori_loop(0, D_MODEL // _LANES, i_body, 0)

        fire_gather(0, 0)

        def pair_body(g2, carry):
            for q in range(2):
                p = g2 * 2 + q

                def drain_e(p=p, q=q):
                    pltpu.make_async_copy(ebuf[q], eout(p - 2), ew_sem[q]).wait()

                pl.when(g2 > 0)(drain_e)
                expand(p, q)
                pltpu.async_copy(ebuf[q], eout(p), ew_sem[q])

                wait_gather(q)
                pltpu.async_copy(gbuf[q], gout(p), gw_sem[q])

                def drain_g(p=p, q=q):
                    pltpu.make_async_copy(
                        gbuf[1 - q], gout(p - 1), gw_sem[1 - q]).wait()

                def nxt(p=p, q=q):
                    fire_gather(p + 1, 1 - q)

                if q == 1:
                    drain_g()
                    pl.when(g2 < _NPAIR // 2 - 1)(nxt)
                else:
                    pl.when(g2 > 0)(drain_g)
                    nxt()
            return carry

        lax.fori_loop(0, _NPAIR // 2, pair_body, 0)
        pltpu.make_async_copy(gbuf[1], gout(_NPAIR - 1), gw_sem[1]).wait()
        pltpu.make_async_copy(ebuf[0], eout(_NPAIR - 2), ew_sem[0]).wait()
        pltpu.make_async_copy(ebuf[1], eout(_NPAIR - 1), ew_sem[1]).wait()

    return sc_embed


_sc_embed = _make_sc_embed()


def kernel(n_flat, embed_table):
    n = jnp.asarray(n_flat, jnp.int32)
    t = jnp.asarray(embed_table, jnp.float32)
    table_rep = jnp.tile(t, (_REP, 1))
    combos = jnp.stack([t[0], t[1] - t[0], t[2] - t[0],
                        t[3] - t[1] - t[2] + t[0]])
    return _sc_embed(n, table_rep, combos)
